# Initial kernel scaffold; baseline (speedup 1.0000x reference)
import jax
import jax.numpy as jnp
from jax.experimental import pallas as pl


def kernel(nodes, to_neighs, u2e, W1, b1, W2, b2, W3, b3):
    raise NotImplementedError("write your pallas kernel here")


# speedup vs baseline: 3.6353x; 3.6353x over previous
import functools

import jax
import jax.numpy as jnp
from jax import lax
from jax.experimental import pallas as pl
from jax.experimental.pallas import tpu as pltpu
from jax.experimental.pallas import tpu_sc as plsc

N = 10000
K = 32
D = 128
V = 100000

NC = 2
NS = 16
NW = NC * NS
E_PER_W = (N * K) // NW
CE = 40
NE = E_PER_W // CE
U_PAD = 10240
U_PER_W = U_PAD // NW
CU = 80
NU = U_PER_W // CU

def _sc_gather_body(table_hbm, eidx_hbm, uidx_hbm, out_e, out_u,
                    eidx_v, uidx_v, ebuf0, ebuf1, ubuf, esem0, esem1, usem):
    wid = lax.axis_index("s") * NC + lax.axis_index("c")
    ebase = wid * E_PER_W
    ubase = wid * U_PER_W

    pltpu.sync_copy(eidx_hbm.at[wid], eidx_v)
    pltpu.sync_copy(uidx_hbm.at[wid], uidx_v)

    ebufs = (ebuf0, ebuf1)
    esems = (esem0, esem1)

    for b in range(2):
        pltpu.async_copy(table_hbm.at[eidx_v.at[b]], ebufs[b], esems[b])

    def body(i, carry):
        for b in range(2):
            c = 2 * i + b
            pltpu.make_async_copy(table_hbm.at[eidx_v.at[c]],
                                  ebufs[b], esems[b]).wait()
            pltpu.sync_copy(ebufs[b], out_e.at[pl.ds(ebase + c * CE, CE)])
            nxt = c + 2

            @pl.when(nxt < NE)
            def _():
                pltpu.async_copy(table_hbm.at[eidx_v.at[nxt]],
                                 ebufs[b], esems[b])
        return carry

    lax.fori_loop(0, NE // 2, body, 0)

    def ubody(c, carry):
        pltpu.async_copy(table_hbm.at[uidx_v.at[c]], ubuf, usem).wait()
        pltpu.sync_copy(ubuf, out_u.at[pl.ds(ubase + c * CU, CU)])
        return carry

    lax.fori_loop(0, NU, ubody, 0)


@functools.cache
def _sc_gather():
    mesh = plsc.VectorSubcoreMesh(core_axis_name="c", subcore_axis_name="s")
    return pl.kernel(
        _sc_gather_body,
        mesh=mesh,
        out_type=(
            jax.ShapeDtypeStruct((N * K, D), jnp.float32),
            jax.ShapeDtypeStruct((U_PAD, D), jnp.float32),
        ),
        scratch_types=[
            pltpu.VMEM((NE, CE), jnp.int32),
            pltpu.VMEM((NU, CU), jnp.int32),
            pltpu.VMEM((CE, D), jnp.float32),
            pltpu.VMEM((CE, D), jnp.float32),
            pltpu.VMEM((CU, D), jnp.float32),
            pltpu.SemaphoreType.DMA,
            pltpu.SemaphoreType.DMA,
            pltpu.SemaphoreType.DMA,
        ],
    )


BN = 200


def _tc_body(e_ref, u_ref, w1a_ref, w1b_ref, w2_ref, w3_ref, b1_ref, b2_ref,
             o_ref):
    x = e_ref[...]
    u = u_ref[...]
    hu = jnp.dot(u, w1b_ref[...], preferred_element_type=jnp.float32)
    hu = hu + b1_ref[...]

    x2 = x.reshape(K * BN, D)
    t1 = jnp.dot(x2, w1a_ref[...], preferred_element_type=jnp.float32)
    h1 = jnp.maximum(t1.reshape(K, BN, D) + hu[None], 0.0)
    t2 = jnp.dot(h1.reshape(K * BN, D), w2_ref[...],
                 preferred_element_type=jnp.float32)
    h2 = jnp.maximum(t2.reshape(K, BN, D) + b2_ref[...][None], 0.0)
    s = jnp.sum(h2 * w3_ref[...][None], axis=2, keepdims=True)
    m = jnp.max(s, axis=0, keepdims=True)
    e = jnp.exp(s - m)
    den = jnp.sum(e, axis=0)
    out = jnp.sum(e * x, axis=0) / den
    o_ref[...] = out


def _tc_mlp(e3, urep, W1a, W1b, W2, w3t, b1, b2):
    grid = (N // BN,)
    return pl.pallas_call(
        _tc_body,
        grid=grid,
        in_specs=[
            pl.BlockSpec((K, BN, D), lambda i: (0, i, 0)),
            pl.BlockSpec((BN, D), lambda i: (i, 0)),
            pl.BlockSpec((D, D), lambda i: (0, 0)),
            pl.BlockSpec((D, D), lambda i: (0, 0)),
            pl.BlockSpec((D, D), lambda i: (0, 0)),
            pl.BlockSpec((1, D), lambda i: (0, 0)),
            pl.BlockSpec((1, D), lambda i: (0, 0)),
            pl.BlockSpec((1, D), lambda i: (0, 0)),
        ],
        out_specs=pl.BlockSpec((BN, D), lambda i: (i, 0)),
        out_shape=jax.ShapeDtypeStruct((N, D), jnp.float32),
    )(e3, urep, W1a, W1b, W2, w3t, b1, b2)


def kernel(nodes, to_neighs, u2e, W1, b1, W2, b2, W3, b3):
    nodes = nodes.astype(jnp.int32)
    to_neighs = to_neighs.astype(jnp.int32)
    eidx = to_neighs.T.reshape(NW, NE, CE)
    uidx = jnp.concatenate(
        [nodes, jnp.zeros((U_PAD - N,), jnp.int32)]).reshape(NW, NU, CU)
    rows_e, rows_u = _sc_gather()(u2e, eidx, uidx)
    e3 = rows_e.reshape(K, N, D)
    out = _tc_mlp(e3, rows_u, W1[:D], W1[D:], W2, W3.T,
                  b1.reshape(1, D), b2.reshape(1, D))
    return out
